# baseline (speedup 1.0000x reference)
import functools

import jax
import jax.numpy as jnp
from jax import lax
from jax.experimental import pallas as pl
from jax.experimental.pallas import tpu as pltpu
from jax.experimental.pallas import tpu_sc as plsc

_B = 8
_R = 29
_K = 2048
_H = 768
_K0 = 2016
_KB = _K0 // 4
_KS = _K - _K0
_CH = 16
_NCH = _KS // _CH
_PB = 2
_NL = 16
_NV = _H // _NL


def _best_of(sims, base):
    lbest = jnp.max(sims, axis=1)
    kiota = lax.broadcasted_iota(jnp.int32, (_B, _KB), 1)
    lidx = jnp.min(jnp.where(sims == lbest[:, None], kiota, _KB),
                   axis=1) + base
    return lbest, lidx


def _tc_body(q_ref, te0_ref, te1_ref, te2_ref, te3_ref, sim_ref, idx_ref):
    q = q_ref[0]
    qn = q / jnp.maximum(jnp.sqrt(jnp.sum(q * q, axis=1, keepdims=True)), 1e-12)

    def sims_of(te):
        tn = te / jnp.maximum(
            jnp.sqrt(jnp.sum(te * te, axis=1, keepdims=True)), 1e-12)
        return lax.dot_general(qn, tn, (((1,), (1,)), ((), ())),
                               preferred_element_type=jnp.float32)

    best, bidx = _best_of(sims_of(te0_ref[0]), 0)
    for s, ref in enumerate((te1_ref, te2_ref, te3_ref)):
        b, i = _best_of(sims_of(ref[0]), (s + 1) * _KB)
        take = b > best
        best = jnp.where(take, b, best)
        bidx = jnp.where(take, i, bidx)
    sim_ref[0, 0] = best
    idx_ref[0, 0] = bidx


def _tc_retrieve(vf_regions, text_embeddings):
    return pl.pallas_call(
        _tc_body,
        grid=(_R,),
        in_specs=[
            pl.BlockSpec((1, _B, _H), lambda r: (r, 0, 0)),
            pl.BlockSpec((1, _KB, _H), lambda r: (r, 0, 0)),
            pl.BlockSpec((1, _KB, _H), lambda r: (r, 1, 0)),
            pl.BlockSpec((1, _KB, _H), lambda r: (r, 2, 0)),
            pl.BlockSpec((1, _KB, _H), lambda r: (r, 3, 0)),
        ],
        out_specs=[
            pl.BlockSpec((1, 1, _B), lambda r: (r, 0, 0)),
            pl.BlockSpec((1, 1, _B), lambda r: (r, 0, 0)),
        ],
        out_shape=[
            jax.ShapeDtypeStruct((_R, 1, _B), jnp.float32),
            jax.ShapeDtypeStruct((_R, 1, _B), jnp.int32),
        ],
        compiler_params=pltpu.CompilerParams(
            dimension_semantics=("arbitrary",),
        ),
    )(vf_regions, text_embeddings, text_embeddings, text_embeddings,
      text_embeddings)


def _inv_norm(nsq):
    i = plsc.bitcast(nsq, jnp.int32)
    y = plsc.bitcast(jnp.int32(0x5F3759DF) - lax.shift_right_arithmetic(i, 1),
                     jnp.float32)
    for _ in range(3):
        y = y * (1.5 - 0.5 * nsq * y * y)
    return jnp.where(nsq < 1e-24, jnp.full((_NL,), 1e12, jnp.float32), y)


def _bf16_rtne(x):
    i = plsc.bitcast(x, jnp.int32)
    lsb = jnp.bitwise_and(lax.shift_right_logical(i, 16), jnp.int32(1))
    r = jnp.bitwise_and(i + jnp.int32(0x7FFF) + lsb, jnp.int32(-65536))
    return plsc.bitcast(r, jnp.float32)


def _lane_sum(x, lanes, tmp):
    for shift in (8, 4, 2, 1):
        tmp[...] = x
        x = x + plsc.load_gather(tmp, [jnp.bitwise_xor(lanes, shift)])
    return x


def _sc_kernel_body(vfr_hbm, te_hbm, sim_hbm, idx_hbm,
                    qbuf, tbuf, osim, oidx, tmp, sem_a, sem_b):
    cid = lax.axis_index("c")
    sid = lax.axis_index("s")
    w = sid * 2 + cid

    @pl.when(w < _R)
    def _work():
        lanes = jnp.arange(_NL, dtype=jnp.int32)

        pltpu.sync_copy(vfr_hbm.at[w], qbuf)

        for b in range(_B):
            def _nsq_body(i, nsq, _b=b):
                t = qbuf[pl.ds(_b * _H + i * _NL, _NL)]
                return nsq + t * t
            nsq = lax.fori_loop(0, _NV, _nsq_body,
                                jnp.zeros((_NL,), jnp.float32))
            rn = _inv_norm(_lane_sum(nsq, lanes, tmp))

            def _scale_body(i, _, _b=b, _rn=rn):
                off = _b * _H + i * _NL
                qbuf[pl.ds(off, _NL)] = _bf16_rtne(qbuf[pl.ds(off, _NL)] * _rn)
                return 0
            lax.fori_loop(0, _NV, _scale_body, 0)

        osim[...] = jnp.full((_NL,), -jnp.inf, jnp.float32)
        oidx[...] = jnp.zeros((_NL,), jnp.int32)

        def _start(ch, par):
            src = te_hbm.at[w, pl.ds(_K0 + ch * _CH, _CH), :]
            sem = sem_a if par == 0 else sem_b
            pltpu.make_async_copy(src, tbuf.at[par], sem).start()

        def _wait(ch, par):
            src = te_hbm.at[w, pl.ds(_K0 + ch * _CH, _CH), :]
            sem = sem_a if par == 0 else sem_b
            pltpu.make_async_copy(src, tbuf.at[par], sem).wait()

        _start(0, 0)

        def _g_body(g, _):
            for par in range(2):
                ch = g * 2 + par
                _wait(ch, par)

                @pl.when(ch + 1 < _NCH)
                def _prefetch(_ch=ch, _par=par):
                    _start(_ch + 1, 1 - _par)

                def _pb_body(pb, _, _par=par, _ch=ch):
                    p0 = pb * _PB

                    def _norms_body(i, ns):
                        ts = [tbuf[_par, p0 + j, pl.ds(i * _NL, _NL)]
                              for j in range(_PB)]
                        return tuple(ns[j] + ts[j] * ts[j]
                                     for j in range(_PB))
                    ns = lax.fori_loop(
                        0, _NV, _norms_body,
                        tuple(jnp.zeros((_NL,), jnp.float32)
                              for _ in range(_PB)))
                    rns = [_inv_norm(_lane_sum(ns[j], lanes, tmp))
                           for j in range(_PB)]

                    def _dot_body(i, accs):
                        qs = [qbuf[pl.ds(b * _H + i * _NL, _NL)]
                              for b in range(_B)]
                        tns = [_bf16_rtne(
                            tbuf[_par, p0 + j, pl.ds(i * _NL, _NL)]
                            * rns[j]) for j in range(_PB)]
                        new = []
                        for j in range(_PB):
                            for b in range(_B):
                                new.append(accs[j * _B + b] + tns[j] * qs[b])
                        return tuple(new)
                    accs = lax.fori_loop(
                        0, _NV, _dot_body,
                        tuple(jnp.zeros((_NL,), jnp.float32)
                              for _ in range(_PB * _B)))

                    for j in range(_PB):
                        cand = jnp.full((_NL,), -jnp.inf, jnp.float32)
                        for b in range(_B):
                            sim_b = _lane_sum(accs[j * _B + b], lanes, tmp)
                            cand = jnp.where(lanes == b, sim_b, cand)
                        row = _K0 + _ch * _CH + p0 + j
                        best = osim[...]
                        take = cand > best
                        osim[...] = jnp.where(take, cand, best)
                        oidx[...] = jnp.where(
                            take, jnp.full((_NL,), row, jnp.int32), oidx[...])
                    return 0

                lax.fori_loop(0, _CH // _PB, _pb_body, 0)
            return 0

        lax.fori_loop(0, _NCH // 2, _g_body, 0)

        pltpu.sync_copy(osim, sim_hbm.at[pl.ds(w * _NL, _NL)])
        pltpu.sync_copy(oidx, idx_hbm.at[pl.ds(w * _NL, _NL)])


def _sc_retrieve(vfr_flat, te):
    mesh = plsc.VectorSubcoreMesh(core_axis_name="c", subcore_axis_name="s")
    return pl.kernel(
        _sc_kernel_body,
        out_type=[
            jax.ShapeDtypeStruct((_R * _NL,), jnp.float32),
            jax.ShapeDtypeStruct((_R * _NL,), jnp.int32),
        ],
        mesh=mesh,
        scratch_types=[
            pltpu.VMEM((_B * _H,), jnp.float32),
            pltpu.VMEM((2, _CH, _H), jnp.float32),
            pltpu.VMEM((_NL,), jnp.float32),
            pltpu.VMEM((_NL,), jnp.int32),
            pltpu.VMEM((_NL,), jnp.float32),
            pltpu.SemaphoreType.DMA,
            pltpu.SemaphoreType.DMA,
        ],
        compiler_params=pltpu.CompilerParams(needs_layout_passes=False),
    )(vfr_flat, te)


@jax.jit
def _retrieve(vf_regions, text_embeddings):
    sc_sim, sc_idx = _sc_retrieve(
        vf_regions.reshape(_R, _B * _H), text_embeddings)
    sc_sim = sc_sim.reshape(_R, _NL)[:, :_B]
    sc_idx = sc_idx.reshape(_R, _NL)[:, :_B]
    tc_sim, tc_idx = _tc_retrieve(vf_regions, text_embeddings)
    tc_sim = tc_sim.reshape(_R, _B)
    tc_idx = tc_idx.reshape(_R, _B)
    take = sc_sim > tc_sim
    best_sim = jnp.where(take, sc_sim, tc_sim)
    best_idx = jnp.where(take, sc_idx, tc_idx)
    return best_sim, best_idx


def kernel(visual_features, text_embeddings):
    vf_regions = jnp.transpose(visual_features[:, 1:1 + _R, :], (1, 0, 2))
    sim, idx = _retrieve(vf_regions, text_embeddings)
    return jnp.transpose(sim, (1, 0)), jnp.transpose(idx, (1, 0))


# speedup vs baseline: 1.2560x; 1.2560x over previous
import jax
import jax.numpy as jnp
from jax import lax
from jax.experimental import pallas as pl
from jax.experimental.pallas import tpu as pltpu

_B = 8
_R = 29
_K = 2048
_H = 768
_KB = 512


def _best_of(sims, base):
    lbest = jnp.max(sims, axis=1)
    kiota = lax.broadcasted_iota(jnp.int32, (_B, _KB), 1)
    lidx = jnp.min(jnp.where(sims == lbest[:, None], kiota, _KB),
                   axis=1) + base
    return lbest, lidx


def _region_body(q_ref, te0_ref, te1_ref, te2_ref, te3_ref, sim_ref, idx_ref):
    q = q_ref[0]
    qn = q / jnp.maximum(jnp.sqrt(jnp.sum(q * q, axis=1, keepdims=True)), 1e-12)

    def sims_of(te):
        tn = te / jnp.maximum(
            jnp.sqrt(jnp.sum(te * te, axis=1, keepdims=True)), 1e-12)
        return lax.dot_general(qn, tn, (((1,), (1,)), ((), ())),
                               preferred_element_type=jnp.float32)

    best, bidx = _best_of(sims_of(te0_ref[0]), 0)
    for s, ref in enumerate((te1_ref, te2_ref, te3_ref)):
        b, i = _best_of(sims_of(ref[0]), (s + 1) * _KB)
        take = b > best
        best = jnp.where(take, b, best)
        bidx = jnp.where(take, i, bidx)
    sim_ref[0, 0] = best
    idx_ref[0, 0] = bidx


@jax.jit
def _retrieve(vf_regions, text_embeddings):
    sim, idx = pl.pallas_call(
        _region_body,
        grid=(_R,),
        in_specs=[
            pl.BlockSpec((1, _B, _H), lambda r: (r, 0, 0)),
            pl.BlockSpec((1, _KB, _H), lambda r: (r, 0, 0)),
            pl.BlockSpec((1, _KB, _H), lambda r: (r, 1, 0)),
            pl.BlockSpec((1, _KB, _H), lambda r: (r, 2, 0)),
            pl.BlockSpec((1, _KB, _H), lambda r: (r, 3, 0)),
        ],
        out_specs=[
            pl.BlockSpec((1, 1, _B), lambda r: (r, 0, 0)),
            pl.BlockSpec((1, 1, _B), lambda r: (r, 0, 0)),
        ],
        out_shape=[
            jax.ShapeDtypeStruct((_R, 1, _B), jnp.float32),
            jax.ShapeDtypeStruct((_R, 1, _B), jnp.int32),
        ],
        compiler_params=pltpu.CompilerParams(
            dimension_semantics=("arbitrary",),
        ),
    )(vf_regions, text_embeddings, text_embeddings, text_embeddings,
      text_embeddings)
    return sim, idx


def kernel(visual_features, text_embeddings):
    vf_regions = jnp.transpose(visual_features[:, 1:1 + _R, :], (1, 0, 2))
    sim, idx = _retrieve(vf_regions, text_embeddings)
    best_sim = jnp.transpose(sim.reshape(_R, _B), (1, 0))
    best_idx = jnp.transpose(idx.reshape(_R, _B), (1, 0))
    return best_sim, best_idx
